# baseline (speedup 1.0000x reference)
import jax
import jax.numpy as jnp
from jax.experimental import pallas as pl

N = 10000
H = 128
T = 12


def _lstm_last(x, W_ih, W_hh, b_ih, b_hh):
    n = x.shape[0]
    h0 = jnp.zeros((n, H), dtype=x.dtype)
    c0 = jnp.zeros((n, H), dtype=x.dtype)

    def step(carry, xt):
        h, c = carry
        gates = xt @ W_ih.T + h @ W_hh.T + b_ih + b_hh
        i, f, g, o = jnp.split(gates, 4, axis=1)
        i = jax.nn.sigmoid(i)
        f = jax.nn.sigmoid(f)
        g = jnp.tanh(g)
        o = jax.nn.sigmoid(o)
        c = f * c + i * g
        h = o * jnp.tanh(c)
        return (h, c), None

    (h, c), _ = jax.lax.scan(step, (h0, c0), jnp.swapaxes(x, 0, 1))
    return h


def _gcn_conv(x, src, dst, ew, W, b):
    n = x.shape[0]
    loop = jnp.arange(n, dtype=src.dtype)
    s = jnp.concatenate([src, loop])
    d = jnp.concatenate([dst, loop])
    w = jnp.concatenate([ew, jnp.ones((n,), dtype=ew.dtype)])
    deg = jnp.zeros((n,), dtype=ew.dtype).at[d].add(w)
    dinv = jnp.where(deg > 0, jax.lax.rsqrt(jnp.maximum(deg, 1e-12)), 0.0)
    norm = dinv[s] * w * dinv[d]
    h = x @ W.T
    msg = h[s] * norm[:, None]
    out = jnp.zeros((n, h.shape[1]), dtype=h.dtype).at[d].add(msg)
    return out + b


def _head_body(x_ref, w_ref, out_ref):
    out_ref[...] = x_ref[...] @ w_ref[...].T


def kernel(x_static, x_dynamic, edge_index, edge_weight, W_ih, W_hh, b_ih, b_hh, W1, b1, W2, b2, Wl, bl):
    dyn = _lstm_last(x_dynamic, W_ih, W_hh, b_ih, b_hh)
    x = jnp.concatenate([x_static, dyn], axis=1)
    src = edge_index[0]
    dst = edge_index[1]
    x = jax.nn.relu(_gcn_conv(x, src, dst, edge_weight, W1, b1))
    x = jax.nn.relu(_gcn_conv(x, src, dst, edge_weight, W2, b2))
    BN = 1000
    y = pl.pallas_call(
        _head_body,
        grid=(N // BN,),
        in_specs=[
            pl.BlockSpec((BN, H), lambda i: (i, 0)),
            pl.BlockSpec((1, H), lambda i: (0, 0)),
        ],
        out_specs=pl.BlockSpec((BN, 1), lambda i: (i, 0)),
        out_shape=jax.ShapeDtypeStruct((N, 1), jnp.float32),
    )(x, Wl)
    return y[:, 0] + bl[0]


# speedup vs baseline: 8.7165x; 8.7165x over previous
import functools

import jax
import jax.numpy as jnp
from jax import lax
from jax.experimental import pallas as pl
from jax.experimental.pallas import tpu as pltpu
import jax.experimental.pallas.tpu_sc as plsc

N = 10000
E = 320000
STATIC = 128
DYN = 16
T = 12
H = 128

PN = 10240
BN = 1280
GRID = PN // BN

NC, NS = 2, 16
NTILE = NC * NS
EPT = E // NTILE
CH = 128
NFULL = EPT // CH
TAILE = EPT - NFULL * CH
RPT = PN // NS
DR = PN // 128
DROW = 8
DNW = DR // DROW

_mesh = plsc.VectorSubcoreMesh(core_axis_name="c", subcore_axis_name="s",
                               num_cores=NC, num_subcores=NS)

def _dot(a, b):
    return jnp.dot(a, b, preferred_element_type=jnp.float32)


@functools.partial(
    pl.kernel,
    out_type=jax.ShapeDtypeStruct((NC, DR, 128), jnp.float32),
    mesh=_mesh,
    compiler_params=pltpu.CompilerParams(needs_layout_passes=False),
    scratch_types=[
        pltpu.VMEM((CH,), jnp.int32),
        pltpu.VMEM((CH,), jnp.float32),
        pltpu.VMEM((CH,), jnp.int32),
        pltpu.VMEM((CH, 128), jnp.float32),
        pltpu.VMEM((TAILE,), jnp.int32),
        pltpu.VMEM((TAILE,), jnp.float32),
        pltpu.VMEM((TAILE,), jnp.int32),
        pltpu.VMEM((TAILE, 128), jnp.float32),
        pltpu.VMEM((DROW, 128), jnp.float32),
        pltpu.VMEM_SHARED((DR, 128), jnp.float32),
    ],
)
def _deg_kernel(dst_hbm, w_hbm, out_hbm, dstv, wv, rowd, rows, dstv_t, wv_t,
                rowd_t, rows_t, zb, accd):
    c = lax.axis_index("c")
    s = lax.axis_index("s")
    wid = c * NS + s
    ebase = wid * EPT

    zero16 = jnp.zeros((16,), jnp.float32)
    iota16 = lax.iota(jnp.int32, 16)

    for i in range(DROW):
        for j in range(8):
            zb[i, pl.ds(j * 16, 16)] = zero16

    @pl.when(s < DNW)
    def _():
        pltpu.sync_copy(zb, accd.at[pl.ds(s * DROW, DROW), :])
    plsc.subcore_barrier()

    def _build(dstv_ref, wv_ref, rowd_ref, rows_ref, n):
        def _g(g, __):
            d16 = dstv_ref[pl.ds(g * 16, 16)]
            rowd_ref[pl.ds(g * 16, 16)] = jnp.right_shift(d16, 7)
            return __
        lax.fori_loop(0, n // 16, _g, None)

        def _e(e, _):
            db = plsc.load_gather(dstv_ref, [jnp.full((16,), e, jnp.int32)])
            wb = plsc.load_gather(wv_ref, [jnp.full((16,), e, jnp.int32)])
            lane = jnp.bitwise_and(db, 127)
            for j in range(8):
                rows_ref[e, pl.ds(j * 16, 16)] = jnp.where(
                    iota16 + j * 16 == lane, wb, 0.0)
            return _
        lax.fori_loop(0, n, _e, None)

    def _chunk(i, _):
        base = ebase + i * CH
        pltpu.sync_copy(dst_hbm.at[pl.ds(base, CH)], dstv)
        pltpu.sync_copy(w_hbm.at[pl.ds(base, CH)], wv)
        _build(dstv, wv, rowd, rows, CH)
        pltpu.sync_copy(rows, accd.at[rowd], add=True)
        return _
    lax.fori_loop(0, NFULL, _chunk, None)

    tbase = ebase + NFULL * CH
    pltpu.sync_copy(dst_hbm.at[pl.ds(tbase, TAILE)], dstv_t)
    pltpu.sync_copy(w_hbm.at[pl.ds(tbase, TAILE)], wv_t)
    _build(dstv_t, wv_t, rowd_t, rows_t, TAILE)
    pltpu.sync_copy(rows_t, accd.at[rowd_t], add=True)

    plsc.subcore_barrier()

    @pl.when(s < DNW)
    def _():
        pltpu.sync_copy(accd.at[pl.ds(s * DROW, DROW), :],
                        out_hbm.at[c, pl.ds(s * DROW, DROW), :])


@functools.partial(
    pl.kernel,
    out_type=jax.ShapeDtypeStruct((NC, PN, H), jnp.float32),
    mesh=_mesh,
    compiler_params=pltpu.CompilerParams(needs_layout_passes=False),
    scratch_types=[
        pltpu.VMEM((CH,), jnp.int32),
        pltpu.VMEM((CH,), jnp.int32),
        pltpu.VMEM((CH,), jnp.float32),
        pltpu.VMEM((TAILE,), jnp.int32),
        pltpu.VMEM((TAILE,), jnp.int32),
        pltpu.VMEM((TAILE,), jnp.float32),
        pltpu.VMEM((CH, H), jnp.float32),
        pltpu.VMEM((TAILE, H), jnp.float32),
        pltpu.VMEM((CH, H), jnp.float32),
        pltpu.VMEM_SHARED((PN, H), jnp.float32),
        pltpu.SemaphoreType.DMA,
    ],
)
def _conv_kernel(h_hbm, src_hbm, dst_hbm, w_hbm, out_hbm, srcv, dstv, wv,
                 srcv_t, dstv_t, wv_t, rows, rows_t, zbuf, acc, sem):
    c = lax.axis_index("c")
    s = lax.axis_index("s")
    wid = c * NS + s
    ebase = wid * EPT

    zero16 = jnp.zeros((16,), jnp.float32)

    def _zb(i, _):
        r = i // (H // 16)
        col = (i % (H // 16)) * 16
        zbuf[r, pl.ds(col, 16)] = zero16
        return _
    lax.fori_loop(0, CH * H // 16, _zb, None)
    for k in range(RPT // CH):
        pltpu.sync_copy(zbuf, acc.at[pl.ds(s * RPT + k * CH, CH), :])
    plsc.subcore_barrier()

    def _scale(rows_ref, wv_ref, n):
        def _e(e, _):
            wb = plsc.load_gather(wv_ref, [jnp.full((16,), e, jnp.int32)])
            for j in range(H // 16):
                rows_ref[e, pl.ds(j * 16, 16)] = (
                    rows_ref[e, pl.ds(j * 16, 16)] * wb)
            return _
        lax.fori_loop(0, n, _e, None)

    def _chunk(i, _):
        base = ebase + i * CH
        pltpu.sync_copy(src_hbm.at[pl.ds(base, CH)], srcv)
        pltpu.sync_copy(dst_hbm.at[pl.ds(base, CH)], dstv)
        pltpu.sync_copy(w_hbm.at[pl.ds(base, CH)], wv)
        pltpu.async_copy(h_hbm.at[srcv], rows, sem).wait()
        _scale(rows, wv, CH)
        pltpu.sync_copy(rows, acc.at[dstv], add=True)
        return _
    lax.fori_loop(0, NFULL, _chunk, None)

    tbase = ebase + NFULL * CH
    pltpu.sync_copy(src_hbm.at[pl.ds(tbase, TAILE)], srcv_t)
    pltpu.sync_copy(dst_hbm.at[pl.ds(tbase, TAILE)], dstv_t)
    pltpu.sync_copy(w_hbm.at[pl.ds(tbase, TAILE)], wv_t)
    pltpu.async_copy(h_hbm.at[srcv_t], rows_t, sem).wait()
    _scale(rows_t, wv_t, TAILE)
    pltpu.sync_copy(rows_t, acc.at[dstv_t], add=True)

    plsc.subcore_barrier()
    pltpu.sync_copy(acc.at[pl.ds(s * RPT, RPT), :],
                    out_hbm.at[c, pl.ds(s * RPT, RPT), :])


def _lstm_body(x_ref, wih_ref, whh_ref, b_ref, dyn_ref):
    x = x_ref[...]
    wih = wih_ref[...]
    whh = whh_ref[...]
    b = b_ref[...]
    h = jnp.zeros((BN, H), jnp.float32)
    cst = jnp.zeros((BN, H), jnp.float32)
    for t in range(T):
        xt = x[:, DYN * t:DYN * (t + 1)]
        gates = _dot(xt, wih) + _dot(h, whh) + b
        ig = jax.nn.sigmoid(gates[:, 0:H])
        fg = jax.nn.sigmoid(gates[:, H:2 * H])
        gg = jnp.tanh(gates[:, 2 * H:3 * H])
        og = jax.nn.sigmoid(gates[:, 3 * H:4 * H])
        cst = fg * cst + ig * gg
        h = og * jnp.tanh(cst)
    dyn_ref[...] = h


def _m1_body(xs_ref, dyn_ref, parts_ref, w1a_ref, w1b_ref, h1s_ref, dinv_ref):
    p = parts_ref[...]
    deg = 1.0 + p[:, 0:1] + p[:, 1:2]
    dinv = jnp.where(deg > 0, lax.rsqrt(jnp.maximum(deg, 1e-12)), 0.0)
    h1 = _dot(xs_ref[...], w1a_ref[...]) + _dot(dyn_ref[...], w1b_ref[...])
    h1s_ref[...] = h1 * dinv
    dinv_ref[...] = dinv


def _m2_body(acc_ref, h1s_ref, dinv_ref, b1_ref, w2_ref, h2s_ref):
    dinv = dinv_ref[...]
    a = acc_ref[0] + acc_ref[1] + h1s_ref[...]
    x1 = jax.nn.relu(a * dinv + b1_ref[...])
    h2s_ref[...] = _dot(x1, w2_ref[...]) * dinv


def _m3_body(acc_ref, h2s_ref, dinv_ref, b2_ref, wl_ref, bl_ref, y_ref):
    dinv = dinv_ref[...]
    a = acc_ref[0] + acc_ref[1] + h2s_ref[...]
    x2 = jax.nn.relu(a * dinv + b2_ref[...])
    y_ref[...] = _dot(x2, wl_ref[...]) + bl_ref[...]


def _row_spec(width):
    return pl.BlockSpec((BN, width), lambda i: (i, 0))


def _full_spec(shape):
    nd = len(shape)
    return pl.BlockSpec(shape, lambda i: (0,) * nd)


def kernel(x_static, x_dynamic, edge_index, edge_weight, W_ih, W_hh, b_ih,
           b_hh, W1, b1, W2, b2, Wl, bl):
    f32 = jnp.float32
    src = edge_index[0]
    dst = edge_index[1]

    x2 = jnp.zeros((PN, T * DYN), f32).at[:N].set(x_dynamic.reshape(N, T * DYN))
    xs = jnp.zeros((PN, STATIC), f32).at[:N].set(x_static)

    wih_t = W_ih.T
    whh_t = W_hh.T
    bg = (b_ih + b_hh).reshape(1, 4 * H)
    w1a_t = W1[:, :STATIC].T
    w1b_t = W1[:, STATIC:].T
    w2_t = W2.T
    wl_t = Wl.T
    b1r = b1.reshape(1, H)
    b2r = b2.reshape(1, H)
    blr = bl.reshape(1, 1)

    deg_parts = _deg_kernel(dst, edge_weight)
    parts_t = deg_parts.reshape(NC, PN).T

    dyn = pl.pallas_call(
        _lstm_body,
        grid=(GRID,),
        in_specs=[_row_spec(T * DYN), _full_spec((DYN, 4 * H)),
                  _full_spec((H, 4 * H)), _full_spec((1, 4 * H))],
        out_specs=_row_spec(H),
        out_shape=jax.ShapeDtypeStruct((PN, H), f32),
    )(x2, wih_t, whh_t, bg)

    h1s, dinv = pl.pallas_call(
        _m1_body,
        grid=(GRID,),
        in_specs=[_row_spec(STATIC), _row_spec(H), _row_spec(2),
                  _full_spec((STATIC, H)), _full_spec((H, H))],
        out_specs=[_row_spec(H), _row_spec(1)],
        out_shape=[jax.ShapeDtypeStruct((PN, H), f32),
                   jax.ShapeDtypeStruct((PN, 1), f32)],
    )(xs, dyn, parts_t, w1a_t, w1b_t)

    acc1 = _conv_kernel(h1s, src, dst, edge_weight)

    h2s = pl.pallas_call(
        _m2_body,
        grid=(GRID,),
        in_specs=[pl.BlockSpec((NC, BN, H), lambda i: (0, i, 0)),
                  _row_spec(H), _row_spec(1), _full_spec((1, H)),
                  _full_spec((H, H))],
        out_specs=_row_spec(H),
        out_shape=jax.ShapeDtypeStruct((PN, H), f32),
    )(acc1, h1s, dinv, b1r, w2_t)

    acc2 = _conv_kernel(h2s, src, dst, edge_weight)

    y = pl.pallas_call(
        _m3_body,
        grid=(GRID,),
        in_specs=[pl.BlockSpec((NC, BN, H), lambda i: (0, i, 0)),
                  _row_spec(H), _row_spec(1), _full_spec((1, H)),
                  _full_spec((H, 1)), _full_spec((1, 1))],
        out_specs=_row_spec(1),
        out_shape=jax.ShapeDtypeStruct((PN, 1), f32),
    )(acc2, h2s, dinv, b2r, wl_t, blr)

    return y[:N, 0]


# speedup vs baseline: 12.0468x; 1.3821x over previous
import functools

import jax
import jax.numpy as jnp
from jax import lax
from jax.experimental import pallas as pl
from jax.experimental.pallas import tpu as pltpu
import jax.experimental.pallas.tpu_sc as plsc

N = 10000
E = 320000
STATIC = 128
DYN = 16
T = 12
H = 128

PN = 10240
BN = 1280
GRID = PN // BN

NC, NS = 2, 16
NTILE = NC * NS
EPT = E // NTILE
CH = 128
NFULL = EPT // CH
TAILE = EPT - NFULL * CH
RPT = PN // NS
DR = PN // 128
DROW = 8
DNW = DR // DROW

_mesh = plsc.VectorSubcoreMesh(core_axis_name="c", subcore_axis_name="s",
                               num_cores=NC, num_subcores=NS)

def _dot(a, b):
    return jnp.dot(a, b, preferred_element_type=jnp.float32)


@functools.partial(
    pl.kernel,
    out_type=jax.ShapeDtypeStruct((NC, DR, 128), jnp.float32),
    mesh=_mesh,
    compiler_params=pltpu.CompilerParams(needs_layout_passes=False),
    scratch_types=[
        pltpu.VMEM((CH,), jnp.int32),
        pltpu.VMEM((CH,), jnp.float32),
        pltpu.VMEM((CH,), jnp.int32),
        pltpu.VMEM((CH, 128), jnp.float32),
        pltpu.VMEM((TAILE,), jnp.int32),
        pltpu.VMEM((TAILE,), jnp.float32),
        pltpu.VMEM((TAILE,), jnp.int32),
        pltpu.VMEM((TAILE, 128), jnp.float32),
        pltpu.VMEM((DROW, 128), jnp.float32),
        pltpu.VMEM_SHARED((DR, 128), jnp.float32),
    ],
)
def _deg_kernel(dst_hbm, w_hbm, out_hbm, dstv, wv, rowd, rows, dstv_t, wv_t,
                rowd_t, rows_t, zb, accd):
    c = lax.axis_index("c")
    s = lax.axis_index("s")
    wid = c * NS + s
    ebase = wid * EPT

    zero16 = jnp.zeros((16,), jnp.float32)
    iota16 = lax.iota(jnp.int32, 16)

    for i in range(DROW):
        for j in range(8):
            zb[i, pl.ds(j * 16, 16)] = zero16

    @pl.when(s < DNW)
    def _():
        pltpu.sync_copy(zb, accd.at[pl.ds(s * DROW, DROW), :])
    plsc.subcore_barrier()

    def _build(dstv_ref, wv_ref, rowd_ref, rows_ref, n):
        def _g(g, __):
            d16 = dstv_ref[pl.ds(g * 16, 16)]
            rowd_ref[pl.ds(g * 16, 16)] = jnp.right_shift(d16, 7)
            return __
        lax.fori_loop(0, n // 16, _g, None)

        def _e(e, _):
            db = plsc.load_gather(dstv_ref, [jnp.full((16,), e, jnp.int32)])
            wb = plsc.load_gather(wv_ref, [jnp.full((16,), e, jnp.int32)])
            lane = jnp.bitwise_and(db, 127)
            for j in range(8):
                rows_ref[e, pl.ds(j * 16, 16)] = jnp.where(
                    iota16 + j * 16 == lane, wb, 0.0)
            return _
        lax.fori_loop(0, n, _e, None)

    def _chunk(i, _):
        base = ebase + i * CH
        pltpu.sync_copy(dst_hbm.at[pl.ds(base, CH)], dstv)
        pltpu.sync_copy(w_hbm.at[pl.ds(base, CH)], wv)
        _build(dstv, wv, rowd, rows, CH)
        pltpu.sync_copy(rows, accd.at[rowd], add=True)
        return _
    lax.fori_loop(0, NFULL, _chunk, None)

    tbase = ebase + NFULL * CH
    pltpu.sync_copy(dst_hbm.at[pl.ds(tbase, TAILE)], dstv_t)
    pltpu.sync_copy(w_hbm.at[pl.ds(tbase, TAILE)], wv_t)
    _build(dstv_t, wv_t, rowd_t, rows_t, TAILE)
    pltpu.sync_copy(rows_t, accd.at[rowd_t], add=True)

    plsc.subcore_barrier()

    @pl.when(s < DNW)
    def _():
        pltpu.sync_copy(accd.at[pl.ds(s * DROW, DROW), :],
                        out_hbm.at[c, pl.ds(s * DROW, DROW), :])


CCH = 120
NCH = 84
EPTP = NCH * CCH
NBUF = 3
NSET = 6
NSTEP = NCH // NSET


@functools.partial(
    pl.kernel,
    out_type=jax.ShapeDtypeStruct((NC, PN, H), jnp.float32),
    mesh=_mesh,
    compiler_params=pltpu.CompilerParams(needs_layout_passes=False),
    scratch_types=[
        [pltpu.VMEM((CCH,), jnp.int32) for _ in range(NSET)],
        [pltpu.VMEM((CCH,), jnp.int32) for _ in range(NSET)],
        [pltpu.VMEM((CCH,), jnp.float32) for _ in range(NSET)],
        [pltpu.VMEM((CCH, H), jnp.float32) for _ in range(NBUF)],
        pltpu.VMEM_SHARED((PN, H), jnp.float32),
        [pltpu.SemaphoreType.DMA for _ in range(NSET)],
        [pltpu.SemaphoreType.DMA for _ in range(NBUF)],
        [pltpu.SemaphoreType.DMA for _ in range(NBUF)],
    ],
)
def _conv_kernel(h_hbm, src_hbm, dst_hbm, w_hbm, out_hbm, srcs, dsts, ws,
                 rows, acc, isem, gsem, ssem):
    c = lax.axis_index("c")
    s = lax.axis_index("s")
    wid = c * NS + s

    zero16 = jnp.zeros((16,), jnp.float32)

    def _zb(i, _):
        r = i // (H // 16)
        col = (i % (H // 16)) * 16
        rows[0][r, pl.ds(col, 16)] = zero16
        return _
    lax.fori_loop(0, CCH * H // 16, _zb, None)
    for off, sz in ((0, 120), (120, 120), (240, 120), (360, 120), (480, 120),
                    (600, 40)):
        pltpu.sync_copy(rows[0].at[pl.ds(0, sz), :],
                        acc.at[pl.ds(s * RPT + off, sz), :])
    plsc.subcore_barrier()

    def _idx_start(ci, si):
        pltpu.async_copy(src_hbm.at[wid, ci], srcs[si], isem[si])
        pltpu.async_copy(dst_hbm.at[wid, ci], dsts[si], isem[si])
        pltpu.async_copy(w_hbm.at[wid, ci], ws[si], isem[si])

    def _idx_wait(ci, si):
        pltpu.make_async_copy(src_hbm.at[wid, ci], srcs[si], isem[si]).wait()
        pltpu.make_async_copy(dst_hbm.at[wid, ci], dsts[si], isem[si]).wait()
        pltpu.make_async_copy(w_hbm.at[wid, ci], ws[si], isem[si]).wait()

    def _scale(b, si):
        def _e(e, _):
            wb = plsc.load_gather(ws[si], [jnp.full((16,), e, jnp.int32)])
            for j in range(H // 16):
                rows[b][e, pl.ds(j * 16, 16)] = (
                    rows[b][e, pl.ds(j * 16, 16)] * wb)
            return _
        lax.fori_loop(0, CCH, _e, None)

    for j in range(NSET):
        _idx_start(j, j)
    for b in range(NBUF):
        _idx_wait(b, b)
        pltpu.async_copy(h_hbm.at[srcs[b]], rows[b], gsem[b])

    def _step(m, _):
        a = m * NSET

        def _P(j):
            b = j % NBUF
            ci = a + j
            pltpu.make_async_copy(h_hbm.at[srcs[j]], rows[b], gsem[b]).wait()
            _scale(b, j)
            pltpu.async_copy(rows[b], acc.at[dsts[j]], ssem[b], add=True)

        def _R(j, cj):
            j2 = j % NSET
            b2 = j2 % NBUF
            pltpu.make_async_copy(rows[b2], acc.at[dsts[j2]], ssem[b2]).wait()

            @pl.when(cj + NSET < NCH)
            def _():
                _idx_start(cj + NSET, j2)

            @pl.when(cj + NBUF < NCH)
            def _():
                j3 = (j2 + NBUF) % NSET
                _idx_wait(cj + NBUF, j3)
                pltpu.async_copy(h_hbm.at[srcs[j3]], rows[b2], gsem[b2])

        _P(0)

        @pl.when(m > 0)
        def _():
            _R(NSET - 1, a - 1)
        _P(1)
        _R(0, a + 0)
        _P(2)
        _R(1, a + 1)
        _P(3)
        _R(2, a + 2)
        _P(4)
        _R(3, a + 3)
        _P(5)
        _R(4, a + 4)
        return _
    lax.fori_loop(0, NSTEP, _step, None)

    pltpu.make_async_copy(rows[2], acc.at[dsts[5]], ssem[2]).wait()

    plsc.subcore_barrier()
    pltpu.sync_copy(acc.at[pl.ds(s * RPT, RPT), :],
                    out_hbm.at[c, pl.ds(s * RPT, RPT), :])


def _lstm_body(x_ref, wih_ref, whh_ref, b_ref, dyn_ref):
    x = x_ref[...]
    wih = wih_ref[...]
    whh = whh_ref[...]
    b = b_ref[...]
    h = jnp.zeros((BN, H), jnp.float32)
    cst = jnp.zeros((BN, H), jnp.float32)
    for t in range(T):
        xt = x[:, DYN * t:DYN * (t + 1)]
        gates = _dot(xt, wih) + _dot(h, whh) + b
        ig = jax.nn.sigmoid(gates[:, 0:H])
        fg = jax.nn.sigmoid(gates[:, H:2 * H])
        gg = jnp.tanh(gates[:, 2 * H:3 * H])
        og = jax.nn.sigmoid(gates[:, 3 * H:4 * H])
        cst = fg * cst + ig * gg
        h = og * jnp.tanh(cst)
    dyn_ref[...] = h


def _m1_body(xs_ref, dyn_ref, parts_ref, w1a_ref, w1b_ref, h1s_ref, dinv_ref):
    p = parts_ref[...]
    deg = 1.0 + p[:, 0:1] + p[:, 1:2]
    dinv = jnp.where(deg > 0, lax.rsqrt(jnp.maximum(deg, 1e-12)), 0.0)
    h1 = _dot(xs_ref[...], w1a_ref[...]) + _dot(dyn_ref[...], w1b_ref[...])
    h1s_ref[...] = h1 * dinv
    dinv_ref[...] = dinv


def _m2_body(acc_ref, h1s_ref, dinv_ref, b1_ref, w2_ref, h2s_ref):
    dinv = dinv_ref[...]
    a = acc_ref[0] + acc_ref[1] + h1s_ref[...]
    x1 = jax.nn.relu(a * dinv + b1_ref[...])
    h2s_ref[...] = _dot(x1, w2_ref[...]) * dinv


def _m3_body(acc_ref, h2s_ref, dinv_ref, b2_ref, wl_ref, bl_ref, y_ref):
    dinv = dinv_ref[...]
    a = acc_ref[0] + acc_ref[1] + h2s_ref[...]
    x2 = jax.nn.relu(a * dinv + b2_ref[...])
    y_ref[...] = _dot(x2, wl_ref[...]) + bl_ref[...]


def _row_spec(width):
    return pl.BlockSpec((BN, width), lambda i: (i, 0))


def _full_spec(shape):
    nd = len(shape)
    return pl.BlockSpec(shape, lambda i: (0,) * nd)


def kernel(x_static, x_dynamic, edge_index, edge_weight, W_ih, W_hh, b_ih,
           b_hh, W1, b1, W2, b2, Wl, bl):
    f32 = jnp.float32
    src = edge_index[0]
    dst = edge_index[1]

    x2 = jnp.zeros((PN, T * DYN), f32).at[:N].set(x_dynamic.reshape(N, T * DYN))
    xs = jnp.zeros((PN, STATIC), f32).at[:N].set(x_static)

    wih_t = W_ih.T
    whh_t = W_hh.T
    bg = (b_ih + b_hh).reshape(1, 4 * H)
    w1a_t = W1[:, :STATIC].T
    w1b_t = W1[:, STATIC:].T
    w2_t = W2.T
    wl_t = Wl.T
    b1r = b1.reshape(1, H)
    b2r = b2.reshape(1, H)
    blr = bl.reshape(1, 1)

    pad = ((0, 0), (0, EPTP - EPT))
    src_p = jnp.pad(src.reshape(NTILE, EPT), pad).reshape(NTILE, NCH, CCH)
    dst_p = jnp.pad(dst.reshape(NTILE, EPT), pad).reshape(NTILE, NCH, CCH)
    w_p = jnp.pad(edge_weight.reshape(NTILE, EPT), pad).reshape(NTILE, NCH, CCH)

    deg_parts = _deg_kernel(dst, edge_weight)
    parts_t = deg_parts.reshape(NC, PN).T

    dyn = pl.pallas_call(
        _lstm_body,
        grid=(GRID,),
        in_specs=[_row_spec(T * DYN), _full_spec((DYN, 4 * H)),
                  _full_spec((H, 4 * H)), _full_spec((1, 4 * H))],
        out_specs=_row_spec(H),
        out_shape=jax.ShapeDtypeStruct((PN, H), f32),
    )(x2, wih_t, whh_t, bg)

    h1s, dinv = pl.pallas_call(
        _m1_body,
        grid=(GRID,),
        in_specs=[_row_spec(STATIC), _row_spec(H), _row_spec(2),
                  _full_spec((STATIC, H)), _full_spec((H, H))],
        out_specs=[_row_spec(H), _row_spec(1)],
        out_shape=[jax.ShapeDtypeStruct((PN, H), f32),
                   jax.ShapeDtypeStruct((PN, 1), f32)],
    )(xs, dyn, parts_t, w1a_t, w1b_t)

    acc1 = _conv_kernel(h1s, src_p, dst_p, w_p)

    h2s = pl.pallas_call(
        _m2_body,
        grid=(GRID,),
        in_specs=[pl.BlockSpec((NC, BN, H), lambda i: (0, i, 0)),
                  _row_spec(H), _row_spec(1), _full_spec((1, H)),
                  _full_spec((H, H))],
        out_specs=_row_spec(H),
        out_shape=jax.ShapeDtypeStruct((PN, H), f32),
    )(acc1, h1s, dinv, b1r, w2_t)

    acc2 = _conv_kernel(h2s, src_p, dst_p, w_p)

    y = pl.pallas_call(
        _m3_body,
        grid=(GRID,),
        in_specs=[pl.BlockSpec((NC, BN, H), lambda i: (0, i, 0)),
                  _row_spec(H), _row_spec(1), _full_spec((1, H)),
                  _full_spec((H, 1)), _full_spec((1, 1))],
        out_specs=_row_spec(1),
        out_shape=jax.ShapeDtypeStruct((PN, 1), f32),
    )(acc2, h2s, dinv, b2r, wl_t, blr)

    return y[:N, 0]


# speedup vs baseline: 14.1940x; 1.1782x over previous
import functools

import jax
import jax.numpy as jnp
from jax import lax
from jax.experimental import pallas as pl
from jax.experimental.pallas import tpu as pltpu
import jax.experimental.pallas.tpu_sc as plsc

N = 10000
E = 320000
STATIC = 128
DYN = 16
T = 12
H = 128

PN = 10240
BN = 1280
GRID = PN // BN

NC, NS = 2, 16
NTILE = NC * NS
EPT = E // NTILE
CH = 128
NFULL = EPT // CH
TAILE = EPT - NFULL * CH
RPT = PN // NS
DR = PN // 128
DROW = 8
DNW = DR // DROW

_mesh = plsc.VectorSubcoreMesh(core_axis_name="c", subcore_axis_name="s",
                               num_cores=NC, num_subcores=NS)

def _dot(a, b):
    return jnp.dot(a, b, preferred_element_type=jnp.float32)


_DCH = 120
_DNCH = 84
_GOFF = (0, 16, 32, 48, 64, 80, 96, 104)


@functools.partial(
    pl.kernel,
    out_type=jax.ShapeDtypeStruct((NC, DR, 128), jnp.float32),
    mesh=_mesh,
    compiler_params=pltpu.CompilerParams(needs_layout_passes=False),
    scratch_types=[
        [pltpu.VMEM((_DCH,), jnp.int32) for _ in range(2)],
        [pltpu.VMEM((_DCH,), jnp.float32) for _ in range(2)],
        [pltpu.VMEM((_DCH,), jnp.int32) for _ in range(2)],
        [pltpu.VMEM((_DCH,), jnp.int32) for _ in range(2)],
        [pltpu.VMEM((_DCH, 128), jnp.float32) for _ in range(2)],
        pltpu.VMEM_SHARED((DR, 128), jnp.float32),
        [pltpu.SemaphoreType.DMA for _ in range(2)],
        [pltpu.SemaphoreType.DMA for _ in range(2)],
    ],
)
def _deg_kernel(dst_hbm, w_hbm, out_hbm, dsts, ws, rowds, prevs, rows, accd,
                isem, ssem):
    c = lax.axis_index("c")
    s = lax.axis_index("s")
    wid = c * NS + s

    zero16f = jnp.zeros((16,), jnp.float32)
    zero16i = jnp.zeros((16,), jnp.int32)
    iota16 = lax.iota(jnp.int32, 16)

    for b in range(2):
        def _z(i, _, b=b):
            r = i // 8
            col = (i % 8) * 16
            rows[b][r, pl.ds(col, 16)] = zero16f
            return _
        lax.fori_loop(0, _DCH * 8, _z, None)
        for off in _GOFF:
            prevs[b][pl.ds(off, 16)] = zero16i

    @pl.when(s < DNW)
    def _():
        pltpu.sync_copy(rows[0].at[pl.ds(0, DROW), :],
                        accd.at[pl.ds(s * DROW, DROW), :])
    plsc.subcore_barrier()

    def _idx_start(ci, b):
        pltpu.async_copy(dst_hbm.at[wid, ci], dsts[b], isem[b])
        pltpu.async_copy(w_hbm.at[wid, ci], ws[b], isem[b])

    def _idx_wait(ci, b):
        pltpu.make_async_copy(dst_hbm.at[wid, ci], dsts[b], isem[b]).wait()
        pltpu.make_async_copy(w_hbm.at[wid, ci], ws[b], isem[b]).wait()

    for b in range(2):
        _idx_start(b, b)

    def _step(m, _):
        for b in range(2):
            ci = 2 * m + b
            _idx_wait(ci, b)

            @pl.when(m > 0)
            def _(b=b):
                pltpu.make_async_copy(rows[b], accd.at[rowds[b]],
                                      ssem[b]).wait()
            for off in _GOFF:
                sl = pl.ds(off, 16)
                row16 = iota16 + off
                old = prevs[b][sl]
                plsc.store_scatter(rows[b], [row16, old], zero16f)
                d16 = dsts[b][sl]
                w16 = ws[b][sl]
                l16 = jnp.bitwise_and(d16, 127)
                plsc.store_scatter(rows[b], [row16, l16], w16)
                prevs[b][sl] = l16
                rowds[b][sl] = jnp.right_shift(d16, 7)
            pltpu.async_copy(rows[b], accd.at[rowds[b]], ssem[b], add=True)

            @pl.when(ci + 2 < _DNCH)
            def _(b=b, ci=ci):
                _idx_start(ci + 2, b)
        return _
    lax.fori_loop(0, _DNCH // 2, _step, None)

    for b in range(2):
        pltpu.make_async_copy(rows[b], accd.at[rowds[b]], ssem[b]).wait()

    plsc.subcore_barrier()

    @pl.when(s < DNW)
    def _():
        pltpu.sync_copy(accd.at[pl.ds(s * DROW, DROW), :],
                        out_hbm.at[c, pl.ds(s * DROW, DROW), :])


CCH = 120
NCH = 84
EPTP = NCH * CCH
NBUF = 3
NSET = 6
NSTEP = NCH // NSET


@functools.partial(
    pl.kernel,
    out_type=jax.ShapeDtypeStruct((NC, PN, H), jnp.float32),
    mesh=_mesh,
    compiler_params=pltpu.CompilerParams(needs_layout_passes=False),
    scratch_types=[
        [pltpu.VMEM((CCH,), jnp.int32) for _ in range(NSET)],
        [pltpu.VMEM((CCH,), jnp.int32) for _ in range(NSET)],
        [pltpu.VMEM((CCH,), jnp.float32) for _ in range(NSET)],
        [pltpu.VMEM((CCH, H), jnp.float32) for _ in range(NBUF)],
        pltpu.VMEM_SHARED((PN, H), jnp.float32),
        [pltpu.SemaphoreType.DMA for _ in range(NSET)],
        [pltpu.SemaphoreType.DMA for _ in range(NBUF)],
        [pltpu.SemaphoreType.DMA for _ in range(NBUF)],
    ],
)
def _conv_kernel(h_hbm, src_hbm, dst_hbm, w_hbm, out_hbm, srcs, dsts, ws,
                 rows, acc, isem, gsem, ssem):
    c = lax.axis_index("c")
    s = lax.axis_index("s")
    wid = c * NS + s

    zero16 = jnp.zeros((16,), jnp.float32)

    def _zb(i, _):
        r = i // (H // 16)
        col = (i % (H // 16)) * 16
        rows[0][r, pl.ds(col, 16)] = zero16
        return _
    lax.fori_loop(0, CCH * H // 16, _zb, None)
    for off, sz in ((0, 120), (120, 120), (240, 120), (360, 120), (480, 120),
                    (600, 40)):
        pltpu.sync_copy(rows[0].at[pl.ds(0, sz), :],
                        acc.at[pl.ds(s * RPT + off, sz), :])
    plsc.subcore_barrier()

    def _idx_start(ci, si):
        pltpu.async_copy(src_hbm.at[wid, ci], srcs[si], isem[si])
        pltpu.async_copy(dst_hbm.at[wid, ci], dsts[si], isem[si])
        pltpu.async_copy(w_hbm.at[wid, ci], ws[si], isem[si])

    def _idx_wait(ci, si):
        pltpu.make_async_copy(src_hbm.at[wid, ci], srcs[si], isem[si]).wait()
        pltpu.make_async_copy(dst_hbm.at[wid, ci], dsts[si], isem[si]).wait()
        pltpu.make_async_copy(w_hbm.at[wid, ci], ws[si], isem[si]).wait()

    def _scale(b, si):
        def _e(e, _):
            wb = plsc.load_gather(ws[si], [jnp.full((16,), e, jnp.int32)])
            for j in range(H // 16):
                rows[b][e, pl.ds(j * 16, 16)] = (
                    rows[b][e, pl.ds(j * 16, 16)] * wb)
            return _
        lax.fori_loop(0, CCH, _e, None)

    for j in range(NSET):
        _idx_start(j, j)
    for b in range(NBUF):
        _idx_wait(b, b)
        pltpu.async_copy(h_hbm.at[srcs[b]], rows[b], gsem[b])

    def _step(m, _):
        a = m * NSET

        def _P(j):
            b = j % NBUF
            ci = a + j
            pltpu.make_async_copy(h_hbm.at[srcs[j]], rows[b], gsem[b]).wait()
            _scale(b, j)
            pltpu.async_copy(rows[b], acc.at[dsts[j]], ssem[b], add=True)

        def _R(j, cj):
            j2 = j % NSET
            b2 = j2 % NBUF
            pltpu.make_async_copy(rows[b2], acc.at[dsts[j2]], ssem[b2]).wait()

            @pl.when(cj + NSET < NCH)
            def _():
                _idx_start(cj + NSET, j2)

            @pl.when(cj + NBUF < NCH)
            def _():
                j3 = (j2 + NBUF) % NSET
                _idx_wait(cj + NBUF, j3)
                pltpu.async_copy(h_hbm.at[srcs[j3]], rows[b2], gsem[b2])

        _P(0)

        @pl.when(m > 0)
        def _():
            _R(NSET - 1, a - 1)
        _P(1)
        _R(0, a + 0)
        _P(2)
        _R(1, a + 1)
        _P(3)
        _R(2, a + 2)
        _P(4)
        _R(3, a + 3)
        _P(5)
        _R(4, a + 4)
        return _
    lax.fori_loop(0, NSTEP, _step, None)

    pltpu.make_async_copy(rows[2], acc.at[dsts[5]], ssem[2]).wait()

    plsc.subcore_barrier()
    pltpu.sync_copy(acc.at[pl.ds(s * RPT, RPT), :],
                    out_hbm.at[c, pl.ds(s * RPT, RPT), :])


def _lstm_body(x_ref, wih_ref, whh_ref, b_ref, dyn_ref):
    x = x_ref[...]
    wih = wih_ref[...]
    whh = whh_ref[...]
    b = b_ref[...]
    h = jnp.zeros((BN, H), jnp.float32)
    cst = jnp.zeros((BN, H), jnp.float32)
    for t in range(T):
        xt = x[:, DYN * t:DYN * (t + 1)]
        gates = _dot(xt, wih) + _dot(h, whh) + b
        ig = jax.nn.sigmoid(gates[:, 0:H])
        fg = jax.nn.sigmoid(gates[:, H:2 * H])
        gg = jnp.tanh(gates[:, 2 * H:3 * H])
        og = jax.nn.sigmoid(gates[:, 3 * H:4 * H])
        cst = fg * cst + ig * gg
        h = og * jnp.tanh(cst)
    dyn_ref[...] = h


def _m1_body(xs_ref, dyn_ref, parts_ref, w1a_ref, w1b_ref, h1s_ref, dinv_ref):
    p = parts_ref[...]
    deg = 1.0 + p[:, 0:1] + p[:, 1:2]
    dinv = jnp.where(deg > 0, lax.rsqrt(jnp.maximum(deg, 1e-12)), 0.0)
    h1 = _dot(xs_ref[...], w1a_ref[...]) + _dot(dyn_ref[...], w1b_ref[...])
    h1s_ref[...] = h1 * dinv
    dinv_ref[...] = dinv


def _m2_body(acc_ref, h1s_ref, dinv_ref, b1_ref, w2_ref, h2s_ref):
    dinv = dinv_ref[...]
    a = acc_ref[0] + acc_ref[1] + h1s_ref[...]
    x1 = jax.nn.relu(a * dinv + b1_ref[...])
    h2s_ref[...] = _dot(x1, w2_ref[...]) * dinv


def _m3_body(acc_ref, h2s_ref, dinv_ref, b2_ref, wl_ref, bl_ref, y_ref):
    dinv = dinv_ref[...]
    a = acc_ref[0] + acc_ref[1] + h2s_ref[...]
    x2 = jax.nn.relu(a * dinv + b2_ref[...])
    y_ref[...] = _dot(x2, wl_ref[...]) + bl_ref[...]


def _row_spec(width):
    return pl.BlockSpec((BN, width), lambda i: (i, 0))


def _full_spec(shape):
    nd = len(shape)
    return pl.BlockSpec(shape, lambda i: (0,) * nd)


def kernel(x_static, x_dynamic, edge_index, edge_weight, W_ih, W_hh, b_ih,
           b_hh, W1, b1, W2, b2, Wl, bl):
    f32 = jnp.float32
    src = edge_index[0]
    dst = edge_index[1]

    x2 = jnp.zeros((PN, T * DYN), f32).at[:N].set(x_dynamic.reshape(N, T * DYN))
    xs = jnp.zeros((PN, STATIC), f32).at[:N].set(x_static)

    wih_t = W_ih.T
    whh_t = W_hh.T
    bg = (b_ih + b_hh).reshape(1, 4 * H)
    w1a_t = W1[:, :STATIC].T
    w1b_t = W1[:, STATIC:].T
    w2_t = W2.T
    wl_t = Wl.T
    b1r = b1.reshape(1, H)
    b2r = b2.reshape(1, H)
    blr = bl.reshape(1, 1)

    pad = ((0, 0), (0, EPTP - EPT))
    src_p = jnp.pad(src.reshape(NTILE, EPT), pad).reshape(NTILE, NCH, CCH)
    dst_p = jnp.pad(dst.reshape(NTILE, EPT), pad).reshape(NTILE, NCH, CCH)
    w_p = jnp.pad(edge_weight.reshape(NTILE, EPT), pad).reshape(NTILE, NCH, CCH)

    deg_parts = _deg_kernel(dst_p, w_p)
    parts_t = deg_parts.reshape(NC, PN).T

    dyn = pl.pallas_call(
        _lstm_body,
        grid=(GRID,),
        in_specs=[_row_spec(T * DYN), _full_spec((DYN, 4 * H)),
                  _full_spec((H, 4 * H)), _full_spec((1, 4 * H))],
        out_specs=_row_spec(H),
        out_shape=jax.ShapeDtypeStruct((PN, H), f32),
    )(x2, wih_t, whh_t, bg)

    h1s, dinv = pl.pallas_call(
        _m1_body,
        grid=(GRID,),
        in_specs=[_row_spec(STATIC), _row_spec(H), _row_spec(2),
                  _full_spec((STATIC, H)), _full_spec((H, H))],
        out_specs=[_row_spec(H), _row_spec(1)],
        out_shape=[jax.ShapeDtypeStruct((PN, H), f32),
                   jax.ShapeDtypeStruct((PN, 1), f32)],
    )(xs, dyn, parts_t, w1a_t, w1b_t)

    acc1 = _conv_kernel(h1s, src_p, dst_p, w_p)

    h2s = pl.pallas_call(
        _m2_body,
        grid=(GRID,),
        in_specs=[pl.BlockSpec((NC, BN, H), lambda i: (0, i, 0)),
                  _row_spec(H), _row_spec(1), _full_spec((1, H)),
                  _full_spec((H, H))],
        out_specs=_row_spec(H),
        out_shape=jax.ShapeDtypeStruct((PN, H), f32),
    )(acc1, h1s, dinv, b1r, w2_t)

    acc2 = _conv_kernel(h2s, src_p, dst_p, w_p)

    y = pl.pallas_call(
        _m3_body,
        grid=(GRID,),
        in_specs=[pl.BlockSpec((NC, BN, H), lambda i: (0, i, 0)),
                  _row_spec(H), _row_spec(1), _full_spec((1, H)),
                  _full_spec((H, 1)), _full_spec((1, 1))],
        out_specs=_row_spec(1),
        out_shape=jax.ShapeDtypeStruct((PN, 1), f32),
    )(acc2, h2s, dinv, b2r, wl_t, blr)

    return y[:N, 0]


# speedup vs baseline: 14.8528x; 1.0464x over previous
import functools

import jax
import jax.numpy as jnp
from jax import lax
from jax.experimental import pallas as pl
from jax.experimental.pallas import tpu as pltpu
import jax.experimental.pallas.tpu_sc as plsc

N = 10000
E = 320000
STATIC = 128
DYN = 16
T = 12
H = 128

PN = 10240
BN = 1280
GRID = PN // BN

NC, NS = 2, 16
NTILE = NC * NS
EPT = E // NTILE
CH = 128
NFULL = EPT // CH
TAILE = EPT - NFULL * CH
RPT = PN // NS
DR = PN // 128
DROW = 8
DNW = DR // DROW

_mesh = plsc.VectorSubcoreMesh(core_axis_name="c", subcore_axis_name="s",
                               num_cores=NC, num_subcores=NS)

def _dot(a, b):
    return jnp.dot(a, b, preferred_element_type=jnp.float32)


_DCH = 120
_DNCH = 84
_GOFF = (0, 16, 32, 48, 64, 80, 96, 104)


@functools.partial(
    pl.kernel,
    out_type=jax.ShapeDtypeStruct((NC, DR, 128), jnp.float32),
    mesh=_mesh,
    compiler_params=pltpu.CompilerParams(needs_layout_passes=False),
    scratch_types=[
        [pltpu.VMEM((_DCH,), jnp.int32) for _ in range(2)],
        [pltpu.VMEM((_DCH,), jnp.float32) for _ in range(2)],
        [pltpu.VMEM((_DCH,), jnp.int32) for _ in range(2)],
        [pltpu.VMEM((_DCH,), jnp.int32) for _ in range(2)],
        [pltpu.VMEM((_DCH, 128), jnp.float32) for _ in range(2)],
        pltpu.VMEM_SHARED((DR, 128), jnp.float32),
        [pltpu.SemaphoreType.DMA for _ in range(2)],
        [pltpu.SemaphoreType.DMA for _ in range(2)],
    ],
)
def _deg_kernel(dst_hbm, w_hbm, out_hbm, dsts, ws, rowds, prevs, rows, accd,
                isem, ssem):
    c = lax.axis_index("c")
    s = lax.axis_index("s")
    wid = c * NS + s

    zero16f = jnp.zeros((16,), jnp.float32)
    zero16i = jnp.zeros((16,), jnp.int32)
    iota16 = lax.iota(jnp.int32, 16)

    for b in range(2):
        def _z(i, _, b=b):
            r = i // 8
            col = (i % 8) * 16
            rows[b][r, pl.ds(col, 16)] = zero16f
            return _
        lax.fori_loop(0, _DCH * 8, _z, None)
        for off in _GOFF:
            prevs[b][pl.ds(off, 16)] = zero16i

    @pl.when(s < DNW)
    def _():
        pltpu.sync_copy(rows[0].at[pl.ds(0, DROW), :],
                        accd.at[pl.ds(s * DROW, DROW), :])
    plsc.subcore_barrier()

    def _idx_start(ci, b):
        pltpu.async_copy(dst_hbm.at[wid, ci], dsts[b], isem[b])
        pltpu.async_copy(w_hbm.at[wid, ci], ws[b], isem[b])

    def _idx_wait(ci, b):
        pltpu.make_async_copy(dst_hbm.at[wid, ci], dsts[b], isem[b]).wait()
        pltpu.make_async_copy(w_hbm.at[wid, ci], ws[b], isem[b]).wait()

    for b in range(2):
        _idx_start(b, b)

    def _step(m, _):
        for b in range(2):
            ci = 2 * m + b
            _idx_wait(ci, b)

            @pl.when(m > 0)
            def _(b=b):
                pltpu.make_async_copy(rows[b], accd.at[rowds[b]],
                                      ssem[b]).wait()
            for off in _GOFF:
                sl = pl.ds(off, 16)
                row16 = iota16 + off
                old = prevs[b][sl]
                plsc.store_scatter(rows[b], [row16, old], zero16f)
                d16 = dsts[b][sl]
                w16 = ws[b][sl]
                l16 = jnp.bitwise_and(d16, 127)
                plsc.store_scatter(rows[b], [row16, l16], w16)
                prevs[b][sl] = l16
                rowds[b][sl] = jnp.right_shift(d16, 7)
            pltpu.async_copy(rows[b], accd.at[rowds[b]], ssem[b], add=True)

            @pl.when(ci + 2 < _DNCH)
            def _(b=b, ci=ci):
                _idx_start(ci + 2, b)
        return _
    lax.fori_loop(0, _DNCH // 2, _step, None)

    for b in range(2):
        pltpu.make_async_copy(rows[b], accd.at[rowds[b]], ssem[b]).wait()

    plsc.subcore_barrier()

    @pl.when(s < DNW)
    def _():
        pltpu.sync_copy(accd.at[pl.ds(s * DROW, DROW), :],
                        out_hbm.at[c, pl.ds(s * DROW, DROW), :])


CCH = 120
NCH = 84
EPTP = NCH * CCH
NBUF = 3
NSET = 6
NSTEP = NCH // NSET


@functools.partial(
    pl.kernel,
    out_type=jax.ShapeDtypeStruct((NC, PN, H), jnp.float32),
    mesh=_mesh,
    compiler_params=pltpu.CompilerParams(needs_layout_passes=False),
    scratch_types=[
        [pltpu.VMEM((CCH,), jnp.int32) for _ in range(NSET)],
        [pltpu.VMEM((CCH,), jnp.int32) for _ in range(NSET)],
        [pltpu.VMEM((CCH,), jnp.float32) for _ in range(NSET)],
        [pltpu.VMEM((CCH, H), jnp.float32) for _ in range(NBUF)],
        pltpu.VMEM_SHARED((PN, H), jnp.float32),
        [pltpu.SemaphoreType.DMA for _ in range(NSET)],
        [pltpu.SemaphoreType.DMA for _ in range(NBUF)],
        [pltpu.SemaphoreType.DMA for _ in range(NBUF)],
    ],
)
def _conv_kernel(h_hbm, src_hbm, dst_hbm, w_hbm, out_hbm, srcs, dsts, ws,
                 rows, acc, isem, gsem, ssem):
    c = lax.axis_index("c")
    s = lax.axis_index("s")
    wid = c * NS + s

    zero16 = jnp.zeros((16,), jnp.float32)

    def _zb(i, _):
        r = i // (H // 16)
        col = (i % (H // 16)) * 16
        rows[0][r, pl.ds(col, 16)] = zero16
        return _
    lax.fori_loop(0, CCH * H // 16, _zb, None)
    for off, sz in ((0, 120), (120, 120), (240, 120), (360, 120), (480, 120),
                    (600, 40)):
        pltpu.sync_copy(rows[0].at[pl.ds(0, sz), :],
                        acc.at[pl.ds(s * RPT + off, sz), :])
    plsc.subcore_barrier()

    def _idx_start(ci, si):
        pltpu.async_copy(src_hbm.at[wid, ci], srcs[si], isem[si])
        pltpu.async_copy(dst_hbm.at[wid, ci], dsts[si], isem[si])
        pltpu.async_copy(w_hbm.at[wid, ci], ws[si], isem[si])

    def _idx_wait(ci, si):
        pltpu.make_async_copy(src_hbm.at[wid, ci], srcs[si], isem[si]).wait()
        pltpu.make_async_copy(dst_hbm.at[wid, ci], dsts[si], isem[si]).wait()
        pltpu.make_async_copy(w_hbm.at[wid, ci], ws[si], isem[si]).wait()

    def _scale(b, si):
        @plsc.parallel_loop(0, CCH, 1, unroll=4)
        def _e(e):
            wb = plsc.load_gather(ws[si], [jnp.full((16,), e, jnp.int32)])
            for j in range(H // 16):
                rows[b][e, pl.ds(j * 16, 16)] = (
                    rows[b][e, pl.ds(j * 16, 16)] * wb)

    for j in range(NSET):
        _idx_start(j, j)
    for b in range(NBUF):
        _idx_wait(b, b)
        pltpu.async_copy(h_hbm.at[srcs[b]], rows[b], gsem[b])

    def _step(m, _):
        a = m * NSET

        def _P(j):
            b = j % NBUF
            ci = a + j
            pltpu.make_async_copy(h_hbm.at[srcs[j]], rows[b], gsem[b]).wait()
            _scale(b, j)
            pltpu.async_copy(rows[b], acc.at[dsts[j]], ssem[b], add=True)

        def _R(j, cj):
            j2 = j % NSET
            b2 = j2 % NBUF
            pltpu.make_async_copy(rows[b2], acc.at[dsts[j2]], ssem[b2]).wait()

            @pl.when(cj + NSET < NCH)
            def _():
                _idx_start(cj + NSET, j2)

            @pl.when(cj + NBUF < NCH)
            def _():
                j3 = (j2 + NBUF) % NSET
                _idx_wait(cj + NBUF, j3)
                pltpu.async_copy(h_hbm.at[srcs[j3]], rows[b2], gsem[b2])

        _P(0)

        @pl.when(m > 0)
        def _():
            _R(NSET - 1, a - 1)
        _P(1)
        _R(0, a + 0)
        _P(2)
        _R(1, a + 1)
        _P(3)
        _R(2, a + 2)
        _P(4)
        _R(3, a + 3)
        _P(5)
        _R(4, a + 4)
        return _
    lax.fori_loop(0, NSTEP, _step, None)

    pltpu.make_async_copy(rows[2], acc.at[dsts[5]], ssem[2]).wait()

    plsc.subcore_barrier()
    pltpu.sync_copy(acc.at[pl.ds(s * RPT, RPT), :],
                    out_hbm.at[c, pl.ds(s * RPT, RPT), :])


def _lstm_body(x_ref, wih_ref, whh_ref, b_ref, dyn_ref):
    x = x_ref[...]
    wih = wih_ref[...]
    whh = whh_ref[...]
    b = b_ref[...]
    h = jnp.zeros((BN, H), jnp.float32)
    cst = jnp.zeros((BN, H), jnp.float32)
    for t in range(T):
        xt = x[:, DYN * t:DYN * (t + 1)]
        gates = _dot(xt, wih) + _dot(h, whh) + b
        ig = jax.nn.sigmoid(gates[:, 0:H])
        fg = jax.nn.sigmoid(gates[:, H:2 * H])
        gg = jnp.tanh(gates[:, 2 * H:3 * H])
        og = jax.nn.sigmoid(gates[:, 3 * H:4 * H])
        cst = fg * cst + ig * gg
        h = og * jnp.tanh(cst)
    dyn_ref[...] = h


def _m1_body(xs_ref, dyn_ref, parts_ref, w1a_ref, w1b_ref, h1s_ref, dinv_ref):
    p = parts_ref[...]
    deg = 1.0 + p[:, 0:1] + p[:, 1:2]
    dinv = jnp.where(deg > 0, lax.rsqrt(jnp.maximum(deg, 1e-12)), 0.0)
    h1 = _dot(xs_ref[...], w1a_ref[...]) + _dot(dyn_ref[...], w1b_ref[...])
    h1s_ref[...] = h1 * dinv
    dinv_ref[...] = dinv


def _m2_body(acc_ref, h1s_ref, dinv_ref, b1_ref, w2_ref, h2s_ref):
    dinv = dinv_ref[...]
    a = acc_ref[0] + acc_ref[1] + h1s_ref[...]
    x1 = jax.nn.relu(a * dinv + b1_ref[...])
    h2s_ref[...] = _dot(x1, w2_ref[...]) * dinv


def _m3_body(acc_ref, h2s_ref, dinv_ref, b2_ref, wl_ref, bl_ref, y_ref):
    dinv = dinv_ref[...]
    a = acc_ref[0] + acc_ref[1] + h2s_ref[...]
    x2 = jax.nn.relu(a * dinv + b2_ref[...])
    y_ref[...] = _dot(x2, wl_ref[...]) + bl_ref[...]


def _row_spec(width):
    return pl.BlockSpec((BN, width), lambda i: (i, 0))


def _full_spec(shape):
    nd = len(shape)
    return pl.BlockSpec(shape, lambda i: (0,) * nd)


def kernel(x_static, x_dynamic, edge_index, edge_weight, W_ih, W_hh, b_ih,
           b_hh, W1, b1, W2, b2, Wl, bl):
    f32 = jnp.float32
    src = edge_index[0]
    dst = edge_index[1]

    x2 = jnp.zeros((PN, T * DYN), f32).at[:N].set(x_dynamic.reshape(N, T * DYN))
    xs = jnp.zeros((PN, STATIC), f32).at[:N].set(x_static)

    wih_t = W_ih.T
    whh_t = W_hh.T
    bg = (b_ih + b_hh).reshape(1, 4 * H)
    w1a_t = W1[:, :STATIC].T
    w1b_t = W1[:, STATIC:].T
    w2_t = W2.T
    wl_t = Wl.T
    b1r = b1.reshape(1, H)
    b2r = b2.reshape(1, H)
    blr = bl.reshape(1, 1)

    pad = ((0, 0), (0, EPTP - EPT))
    src_p = jnp.pad(src.reshape(NTILE, EPT), pad).reshape(NTILE, NCH, CCH)
    dst_p = jnp.pad(dst.reshape(NTILE, EPT), pad).reshape(NTILE, NCH, CCH)
    w_p = jnp.pad(edge_weight.reshape(NTILE, EPT), pad).reshape(NTILE, NCH, CCH)

    deg_parts = _deg_kernel(dst_p, w_p)
    parts_t = deg_parts.reshape(NC, PN).T

    dyn = pl.pallas_call(
        _lstm_body,
        grid=(GRID,),
        in_specs=[_row_spec(T * DYN), _full_spec((DYN, 4 * H)),
                  _full_spec((H, 4 * H)), _full_spec((1, 4 * H))],
        out_specs=_row_spec(H),
        out_shape=jax.ShapeDtypeStruct((PN, H), f32),
    )(x2, wih_t, whh_t, bg)

    h1s, dinv = pl.pallas_call(
        _m1_body,
        grid=(GRID,),
        in_specs=[_row_spec(STATIC), _row_spec(H), _row_spec(2),
                  _full_spec((STATIC, H)), _full_spec((H, H))],
        out_specs=[_row_spec(H), _row_spec(1)],
        out_shape=[jax.ShapeDtypeStruct((PN, H), f32),
                   jax.ShapeDtypeStruct((PN, 1), f32)],
    )(xs, dyn, parts_t, w1a_t, w1b_t)

    acc1 = _conv_kernel(h1s, src_p, dst_p, w_p)

    h2s = pl.pallas_call(
        _m2_body,
        grid=(GRID,),
        in_specs=[pl.BlockSpec((NC, BN, H), lambda i: (0, i, 0)),
                  _row_spec(H), _row_spec(1), _full_spec((1, H)),
                  _full_spec((H, H))],
        out_specs=_row_spec(H),
        out_shape=jax.ShapeDtypeStruct((PN, H), f32),
    )(acc1, h1s, dinv, b1r, w2_t)

    acc2 = _conv_kernel(h2s, src_p, dst_p, w_p)

    y = pl.pallas_call(
        _m3_body,
        grid=(GRID,),
        in_specs=[pl.BlockSpec((NC, BN, H), lambda i: (0, i, 0)),
                  _row_spec(H), _row_spec(1), _full_spec((1, H)),
                  _full_spec((H, 1)), _full_spec((1, 1))],
        out_specs=_row_spec(1),
        out_shape=jax.ShapeDtypeStruct((PN, 1), f32),
    )(acc2, h2s, dinv, b2r, wl_t, blr)

    return y[:N, 0]
